# Initial kernel scaffold; baseline (speedup 1.0000x reference)
import jax
import jax.numpy as jnp
from jax.experimental import pallas as pl


def kernel(node_features, edge_features, node_index, edge_index, params):
    raise NotImplementedError("write your pallas kernel here")


# speedup vs baseline: 2.6330x; 2.6330x over previous
import functools

import jax
import jax.numpy as jnp
from jax import lax
from jax.experimental import pallas as pl
from jax.experimental.pallas import tpu as pltpu
from jax.experimental.pallas import tpu_sc as plsc

N_NODES = 10000
N_EDGES = 160000
H = 128

NC = 2
NS = 16
NW = NC * NS
CH = 128
NCH = N_EDGES // CH
CH_PER_W = -(-NCH // NW)
NN_PER_S = 624
EPC = N_EDGES // NC
NCH_C = EPC // CH
CH_PER_S = -(-NCH_C // NS)


def _ln(x, g, b):
    mu = jnp.mean(x, axis=-1, keepdims=True)
    var = jnp.mean((x - mu) ** 2, axis=-1, keepdims=True)
    return (x - mu) * jax.lax.rsqrt(var + 1e-5) * g + b


def _silu(x):
    return x * jax.nn.sigmoid(x)


def _node_pre_body(nf, w0, b0, w1, b1, g, beta, w1a, w1b,
                   ne_out, p_out, q_out):
    x = _silu(jnp.dot(nf[...], w0[...], preferred_element_type=jnp.float32) + b0[...])
    x = jnp.dot(x, w1[...], preferred_element_type=jnp.float32) + b1[...]
    ne = _ln(x, g[...], beta[...])
    ne_out[...] = ne
    p_out[...] = jnp.dot(ne, w1a[...], preferred_element_type=jnp.float32)
    q_out[...] = jnp.dot(ne, w1b[...], preferred_element_type=jnp.float32)


def _node_pre(nf, w0, b0, w1, b1, g, beta, w1a, w1b):
    out = jax.ShapeDtypeStruct((N_NODES, H), jnp.float32)
    return pl.pallas_call(
        _node_pre_body,
        out_shape=[out, out, out],
    )(nf, w0, b0, w1, b1, g, beta, w1a, w1b)


def _edge_pre_body(ef, w0, b0, w1, b1, g, beta, w1c, b1c, c_out):
    x = _silu(jnp.dot(ef[...], w0[...], preferred_element_type=jnp.float32) + b0[...])
    x = jnp.dot(x, w1[...], preferred_element_type=jnp.float32) + b1[...]
    e = _ln(x, g[...], beta[...])
    c_out[...] = jnp.dot(e, w1c[...], preferred_element_type=jnp.float32) + b1c[...]


def _edge_pre(ef, w0, b0, w1, b1, g, beta, w1c, b1c):
    R = 4000
    grid = (N_EDGES // R,)
    def rows(i):
        return (i, 0)
    def rep(i):
        return (0, 0)
    return pl.pallas_call(
        _edge_pre_body,
        grid=grid,
        in_specs=[
            pl.BlockSpec((R, 16), rows),
            pl.BlockSpec((16, H), rep), pl.BlockSpec((1, H), rep),
            pl.BlockSpec((H, H), rep), pl.BlockSpec((1, H), rep),
            pl.BlockSpec((1, H), rep), pl.BlockSpec((1, H), rep),
            pl.BlockSpec((H, H), rep), pl.BlockSpec((1, H), rep),
        ],
        out_specs=pl.BlockSpec((R, H), rows),
        out_shape=jax.ShapeDtypeStruct((N_EDGES, H), jnp.float32),
    )(ef, w0, b0, w1, b1, g, beta, w1c, b1c)


def _edge_post_body(h1, w1, b1, g, beta, e_out):
    x = jnp.dot(_silu(h1[...]), w1[...], preferred_element_type=jnp.float32) + b1[...]
    e_out[...] = _ln(x, g[...], beta[...])


def _edge_post(h1, w1, b1, g, beta):
    R = 4000
    grid = (N_EDGES // R,)
    def rows(i):
        return (i, 0)
    def rep(i):
        return (0, 0)
    return pl.pallas_call(
        _edge_post_body,
        grid=grid,
        in_specs=[
            pl.BlockSpec((R, H), rows),
            pl.BlockSpec((H, H), rep), pl.BlockSpec((1, H), rep),
            pl.BlockSpec((1, H), rep), pl.BlockSpec((1, H), rep),
        ],
        out_specs=pl.BlockSpec((R, H), rows),
        out_shape=jax.ShapeDtypeStruct((N_EDGES, H), jnp.float32),
    )(h1, w1, b1, g, beta)


def _node_post_body(ne, agg_ref, wn0a, wn0b, bn0, wn1, bn1, gn, btn,
                    wd0, bd0, wd1, bd1, gd, btd, out):
    agg = agg_ref[...]
    x = _silu(jnp.dot(ne[...], wn0a[...], preferred_element_type=jnp.float32)
              + jnp.dot(agg, wn0b[...], preferred_element_type=jnp.float32)
              + bn0[...])
    x = jnp.dot(x, wn1[...], preferred_element_type=jnp.float32) + bn1[...]
    ne2 = ne[...] + _ln(x, gn[...], btn[...])
    y = _silu(jnp.dot(ne2, wd0[...], preferred_element_type=jnp.float32) + bd0[...])
    y = jnp.dot(y, wd1[...], preferred_element_type=jnp.float32) + bd1[...]
    out[...] = _ln(y, gd[...], btd[...])


def _node_post(ne, agg, wn0a, wn0b, bn0, wn1, bn1, gn, btn,
               wd0, bd0, wd1, bd1, gd, btd):
    return pl.pallas_call(
        _node_post_body,
        out_shape=jax.ShapeDtypeStruct((N_NODES, H), jnp.float32),
    )(ne, agg, wn0a, wn0b, bn0, wn1, bn1, gn, btn,
      wd0, bd0, wd1, bd1, gd, btd)


def _sc_gather(p, q, c, src, dst):
    mesh = plsc.VectorSubcoreMesh(core_axis_name="c", subcore_axis_name="s")

    @functools.partial(
        pl.kernel,
        mesh=mesh,
        out_type=jax.ShapeDtypeStruct((N_EDGES, H), jnp.float32),
        scratch_types=[
            pltpu.VMEM((CH,), jnp.int32),
            pltpu.VMEM((CH,), jnp.int32),
            pltpu.VMEM((CH, H), jnp.float32),
            pltpu.VMEM((CH, H), jnp.float32),
            pltpu.VMEM((CH, H), jnp.float32),
            pltpu.SemaphoreType.DMA,
            pltpu.SemaphoreType.DMA,
            pltpu.SemaphoreType.DMA,
        ],
    )
    def run(p_hbm, q_hbm, c_hbm, si_hbm, di_hbm, h_hbm,
            si_v, di_v, pbuf, qbuf, cbuf, sem1, sem2, sem3):
        wid = lax.axis_index("s") * NC + lax.axis_index("c")

        @pl.loop(0, CH_PER_W)
        def _(i):
            ch = wid + i * NW

            @pl.when(ch < NCH)
            def _():
                base = ch * CH
                pltpu.sync_copy(si_hbm.at[pl.ds(base, CH)], si_v)
                pltpu.sync_copy(di_hbm.at[pl.ds(base, CH)], di_v)
                cp1 = pltpu.async_copy(p_hbm.at[si_v], pbuf, sem1)
                cp2 = pltpu.async_copy(q_hbm.at[di_v], qbuf, sem2)
                cp3 = pltpu.async_copy(c_hbm.at[pl.ds(base, CH)], cbuf, sem3)
                cp1.wait()
                cp2.wait()
                cp3.wait()

                @pl.loop(0, CH)
                def _(r):
                    for j in range(H // 16):
                        sl = pl.ds(j * 16, 16)
                        cbuf[r, sl] = cbuf[r, sl] + pbuf[r, sl] + qbuf[r, sl]

                pltpu.sync_copy(cbuf, h_hbm.at[pl.ds(base, CH)])

    return run(p, q, c, src, dst)


NHALF = N_NODES // NC
NDUMMY = 1024
ROWS_S = 312


def _sc_scatter(e_upd, dst):
    mesh = plsc.VectorSubcoreMesh(core_axis_name="c", subcore_axis_name="s")

    @functools.partial(
        pl.kernel,
        mesh=mesh,
        out_type=jax.ShapeDtypeStruct((N_NODES, H), jnp.float32),
        scratch_types=[
            pltpu.VMEM((CH,), jnp.int32),
            pltpu.VMEM((CH,), jnp.int32),
            pltpu.VMEM((CH, H), jnp.float32),
            pltpu.VMEM((320, H), jnp.float32),
            pltpu.VMEM_SHARED((NHALF + NDUMMY, H), jnp.float32),
            pltpu.SemaphoreType.DMA,
        ],
    )
    def run(e_hbm, di_hbm, out_hbm, di_v, di2_v, ebuf, zbuf, acc, sem):
        cid = lax.axis_index("c")
        sid = lax.axis_index("s")
        lo = cid * NHALF

        @pl.loop(0, 320)
        def _(r):
            for j in range(H // 16):
                zbuf[r, pl.ds(j * 16, 16)] = jnp.zeros((16,), jnp.float32)

        @pl.when(sid < NS - 1)
        def _():
            pltpu.sync_copy(zbuf.at[pl.ds(0, ROWS_S)],
                            acc.at[pl.ds(sid * ROWS_S, ROWS_S)])

        @pl.when(sid == NS - 1)
        def _():
            pltpu.sync_copy(zbuf, acc.at[pl.ds(sid * ROWS_S, 320)])

        plsc.subcore_barrier()

        @pl.loop(0, -(-NCH // NS))
        def _(i):
            ch = sid + i * NS

            @pl.when(ch < NCH)
            def _():
                base = ch * CH
                pltpu.sync_copy(di_hbm.at[pl.ds(base, CH)], di_v)
                cp = pltpu.async_copy(e_hbm.at[pl.ds(base, CH)], ebuf, sem)
                for j in range(CH // 16):
                    sl = pl.ds(j * 16, 16)
                    idx = di_v[sl]
                    rel = idx - lo
                    inr = (rel >= 0) & (rel < NHALF)
                    dummy = NHALF + (idx & (NDUMMY - 1))
                    di2_v[sl] = jnp.where(inr, rel, dummy)
                cp.wait()
                pltpu.sync_copy(ebuf, acc.at[di2_v], add=True)

        plsc.subcore_barrier()

        @pl.when(sid < NS - 1)
        def _():
            pltpu.sync_copy(acc.at[pl.ds(sid * ROWS_S, ROWS_S)],
                            out_hbm.at[pl.ds(lo + sid * ROWS_S, ROWS_S)])

        @pl.when(sid == NS - 1)
        def _():
            pltpu.sync_copy(acc.at[pl.ds(sid * ROWS_S, 320)],
                            out_hbm.at[pl.ds(lo + sid * ROWS_S, 320)])

    return run(e_upd, dst)


def kernel(node_features, edge_features, node_index, edge_index, params):
    nf = node_features.reshape(N_NODES, 128)
    ef = edge_features.reshape(N_EDGES, 16)
    src = edge_index[0].astype(jnp.int32)
    dst = edge_index[1].astype(jnp.int32)

    def lw(p, i):
        W, b = p["layers"][i]
        return W, b.reshape(1, -1)

    def lng(p):
        g, b = p["ln"]
        return g.reshape(1, -1), b.reshape(1, -1)

    we0, be0 = lw(params["embed"], 0)
    we1, be1 = lw(params["embed"], 1)
    ge, bte = lng(params["embed"])

    wee0, bee0 = lw(params["edge_embed"], 0)
    wee1, bee1 = lw(params["edge_embed"], 1)
    gee, btee = lng(params["edge_embed"])

    wm0, bm0 = lw(params["edge_mlp"], 0)
    wm1, bm1 = lw(params["edge_mlp"], 1)
    gm, btm = lng(params["edge_mlp"])
    w1a, w1b, w1c = wm0[0:H], wm0[H:2 * H], wm0[2 * H:3 * H]

    wn0, bn0 = lw(params["node_mlp"], 0)
    wn1, bn1 = lw(params["node_mlp"], 1)
    gn, btn = lng(params["node_mlp"])
    wn0a, wn0b = wn0[0:H], wn0[H:2 * H]

    wd0, bd0 = lw(params["deembed"], 0)
    wd1, bd1 = lw(params["deembed"], 1)
    gd, btd = lng(params["deembed"])

    ne, p_arr, q_arr = _node_pre(nf, we0, be0, we1, be1, ge, bte, w1a, w1b)
    c_arr = _edge_pre(ef, wee0, bee0, wee1, bee1, gee, btee, w1c, bm0)
    h1 = _sc_gather(p_arr, q_arr, c_arr, src, dst)
    e_upd = _edge_post(h1, wm1, bm1, gm, btm)
    agg = _sc_scatter(e_upd, dst)
    out = _node_post(ne, agg, wn0a, wn0b, bn0, wn1, bn1, gn, btn,
                     wd0, bd0, wd1, bd1, gd, btd)
    return out.reshape(1, N_NODES, H)


# speedup vs baseline: 3.8195x; 1.4506x over previous
import functools

import jax
import jax.numpy as jnp
from jax import lax
from jax.experimental import pallas as pl
from jax.experimental.pallas import tpu as pltpu
from jax.experimental.pallas import tpu_sc as plsc

N_NODES = 10000
N_EDGES = 160000
H = 128

NC = 2
NS = 16
NW = NC * NS
CH = 128
NCH = N_EDGES // CH
NCH_PAD = 1280

G_MAX = NCH_PAD // NW
G_NBUF = 3

S_MAX = NCH_PAD // NS

NHALF = N_NODES // NC
NDUMMY = 1024
ROWS_S = 312


def _ln(x, g, b):
    mu = jnp.mean(x, axis=-1, keepdims=True)
    var = jnp.mean((x - mu) ** 2, axis=-1, keepdims=True)
    return (x - mu) * jax.lax.rsqrt(var + 1e-5) * g + b


def _silu(x):
    return x * jax.nn.sigmoid(x)


def _node_pre_body(nf, w0, b0, w1, b1, g, beta, w1a, w1b,
                   ne_out, p_out, q_out):
    x = _silu(jnp.dot(nf[...], w0[...], preferred_element_type=jnp.float32) + b0[...])
    x = jnp.dot(x, w1[...], preferred_element_type=jnp.float32) + b1[...]
    ne = _ln(x, g[...], beta[...])
    ne_out[...] = ne
    p_out[...] = jnp.dot(ne, w1a[...], preferred_element_type=jnp.float32)
    q_out[...] = jnp.dot(ne, w1b[...], preferred_element_type=jnp.float32)


def _node_pre(nf, w0, b0, w1, b1, g, beta, w1a, w1b):
    return pl.pallas_call(
        _node_pre_body,
        out_shape=[jax.ShapeDtypeStruct((N_NODES, H), jnp.float32),
                   jax.ShapeDtypeStruct((N_NODES, H), jnp.float32),
                   jax.ShapeDtypeStruct((N_NODES, H), jnp.float32)],
    )(nf, w0, b0, w1, b1, g, beta, w1a, w1b)


def _edge_fused_body(ef, pg, qg, w0, b0, w1, b1, g, beta, w1c, b1c,
                     wm1, bm1, gm, btm, e_out):
    x = _silu(jnp.dot(ef[...], w0[...], preferred_element_type=jnp.float32) + b0[...])
    x = jnp.dot(x, w1[...], preferred_element_type=jnp.float32) + b1[...]
    e = _ln(x, g[...], beta[...])
    c = jnp.dot(e, w1c[...], preferred_element_type=jnp.float32) + b1c[...]
    h1 = pg[...].astype(jnp.float32) + qg[...].astype(jnp.float32) + c
    y = jnp.dot(_silu(h1), wm1[...], preferred_element_type=jnp.float32) + bm1[...]
    e_out[...] = _ln(y, gm[...], btm[...])


def _edge_fused(ef, pg, qg, w0, b0, w1, b1, g, beta, w1c, b1c, wm1, bm1, gm, btm):
    R = 4000
    grid = (N_EDGES // R,)
    def rows(i):
        return (i, 0)
    def rep(i):
        return (0, 0)
    return pl.pallas_call(
        _edge_fused_body,
        grid=grid,
        in_specs=[
            pl.BlockSpec((R, 16), rows),
            pl.BlockSpec((R, H), rows),
            pl.BlockSpec((R, H), rows),
            pl.BlockSpec((16, H), rep), pl.BlockSpec((1, H), rep),
            pl.BlockSpec((H, H), rep), pl.BlockSpec((1, H), rep),
            pl.BlockSpec((1, H), rep), pl.BlockSpec((1, H), rep),
            pl.BlockSpec((H, H), rep), pl.BlockSpec((1, H), rep),
            pl.BlockSpec((H, H), rep), pl.BlockSpec((1, H), rep),
            pl.BlockSpec((1, H), rep), pl.BlockSpec((1, H), rep),
        ],
        out_specs=pl.BlockSpec((R, H), rows),
        out_shape=jax.ShapeDtypeStruct((N_EDGES, H), jnp.float32),
    )(ef, pg, qg, w0, b0, w1, b1, g, beta, w1c, b1c, wm1, bm1, gm, btm)


def _node_post_body(ne, agg_ref, wn0a, wn0b, bn0, wn1, bn1, gn, btn,
                    wd0, bd0, wd1, bd1, gd, btd, out):
    agg = agg_ref[...]
    x = _silu(jnp.dot(ne[...], wn0a[...], preferred_element_type=jnp.float32)
              + jnp.dot(agg, wn0b[...], preferred_element_type=jnp.float32)
              + bn0[...])
    x = jnp.dot(x, wn1[...], preferred_element_type=jnp.float32) + bn1[...]
    ne2 = ne[...] + _ln(x, gn[...], btn[...])
    y = _silu(jnp.dot(ne2, wd0[...], preferred_element_type=jnp.float32) + bd0[...])
    y = jnp.dot(y, wd1[...], preferred_element_type=jnp.float32) + bd1[...]
    out[...] = _ln(y, gd[...], btd[...])


def _node_post(ne, agg, wn0a, wn0b, bn0, wn1, bn1, gn, btn,
               wd0, bd0, wd1, bd1, gd, btd):
    return pl.pallas_call(
        _node_post_body,
        out_shape=jax.ShapeDtypeStruct((N_NODES, H), jnp.float32),
    )(ne, agg, wn0a, wn0b, bn0, wn1, bn1, gn, btn,
      wd0, bd0, wd1, bd1, gd, btd)


def _sc_gather(p, q, si2, di2):
    mesh = plsc.VectorSubcoreMesh(core_axis_name="c", subcore_axis_name="s")
    obf = jax.ShapeDtypeStruct((N_EDGES, H), jnp.float32)

    @functools.partial(
        pl.kernel,
        mesh=mesh,
        out_type=[obf, obf],
        scratch_types=(
            [pltpu.VMEM((G_MAX, CH), jnp.int32),
             pltpu.VMEM((G_MAX, CH), jnp.int32)]
            + [pltpu.VMEM((CH, H), jnp.float32) for _ in range(2 * G_NBUF)]
            + [pltpu.SemaphoreType.DMA for _ in range(2 * G_NBUF)]
        ),
    )
    def run(p_hbm, q_hbm, si_hbm, di_hbm, pg_hbm, qg_hbm, sidx, didx, *bufs):
        pbuf = bufs[0:G_NBUF]
        qbuf = bufs[G_NBUF:2 * G_NBUF]
        semg = bufs[2 * G_NBUF:3 * G_NBUF]
        sems = bufs[3 * G_NBUF:4 * G_NBUF]

        wid = lax.axis_index("s") * NC + lax.axis_index("c")
        first = wid * G_MAX
        n_ch = jnp.minimum(jnp.maximum(NCH - first, 0), G_MAX)

        pltpu.sync_copy(si_hbm.at[pl.ds(first, G_MAX)], sidx)
        pltpu.sync_copy(di_hbm.at[pl.ds(first, G_MAX)], didx)

        ngroups = -(-G_MAX // G_NBUF)

        @pl.loop(0, ngroups)
        def _(t):
            for b in range(G_NBUF):
                i = t * G_NBUF + b

                @pl.when(i < n_ch)
                def _():
                    pltpu.async_copy(p_hbm.at[sidx.at[i]], pbuf[b], semg[b])
                    pltpu.async_copy(q_hbm.at[didx.at[i]], qbuf[b], semg[b])

            for b in range(G_NBUF):
                i = t * G_NBUF + b

                @pl.when(i < n_ch)
                def _():
                    base = (first + i) * CH
                    pltpu.make_async_copy(p_hbm.at[sidx.at[i]], pbuf[b], semg[b]).wait()
                    pltpu.make_async_copy(q_hbm.at[didx.at[i]], qbuf[b], semg[b]).wait()
                    pltpu.async_copy(pbuf[b], pg_hbm.at[pl.ds(base, CH)], sems[b])
                    pltpu.async_copy(qbuf[b], qg_hbm.at[pl.ds(base, CH)], sems[b])

            for b in range(G_NBUF):
                i = t * G_NBUF + b

                @pl.when(i < n_ch)
                def _():
                    base = (first + i) * CH
                    pltpu.make_async_copy(pbuf[b], pg_hbm.at[pl.ds(base, CH)], sems[b]).wait()
                    pltpu.make_async_copy(qbuf[b], qg_hbm.at[pl.ds(base, CH)], sems[b]).wait()

    return run(p, q, si2, di2)


def _sc_scatter(e_upd, di2):
    mesh = plsc.VectorSubcoreMesh(core_axis_name="c", subcore_axis_name="s")

    @functools.partial(
        pl.kernel,
        mesh=mesh,
        out_type=jax.ShapeDtypeStruct((N_NODES, H), jnp.float32),
        scratch_types=[
            pltpu.VMEM((S_MAX, CH), jnp.int32),
            pltpu.VMEM((S_MAX, CH), jnp.int32),
            pltpu.VMEM((CH, H), jnp.float32),
            pltpu.VMEM((CH, H), jnp.float32),
            pltpu.VMEM_SHARED((NHALF + NDUMMY, H), jnp.float32),
            pltpu.SemaphoreType.DMA,
            pltpu.SemaphoreType.DMA,
        ],
    )
    def run(e_hbm, di_hbm, out_hbm, didx, didx2, eb0, eb1, acc, sem0, sem1):
        cid = lax.axis_index("c")
        sid = lax.axis_index("s")
        lo = cid * NHALF
        first = sid * S_MAX
        n_ch = jnp.minimum(jnp.maximum(NCH - first, 0), S_MAX)

        pltpu.sync_copy(di_hbm.at[pl.ds(first, S_MAX)], didx)

        @pl.loop(0, S_MAX)
        def _(r):
            for j in range(CH // 16):
                sl = pl.ds(j * 16, 16)
                idx = didx[r, sl]
                rel = idx - lo
                inr = (rel >= 0) & (rel < NHALF)
                dummy = NHALF + (idx & (NDUMMY - 1))
                didx2[r, sl] = jnp.where(inr, rel, dummy)

        @pl.loop(0, CH)
        def _(r):
            for j in range(H // 16):
                eb0[r, pl.ds(j * 16, 16)] = jnp.zeros((16,), jnp.float32)

        zb = sid * ROWS_S
        pltpu.sync_copy(eb0, acc.at[pl.ds(zb, CH)])
        pltpu.sync_copy(eb0, acc.at[pl.ds(zb + CH, CH)])

        @pl.when(sid < NS - 1)
        def _():
            pltpu.sync_copy(eb0.at[pl.ds(0, ROWS_S - 2 * CH)],
                            acc.at[pl.ds(zb + 2 * CH, ROWS_S - 2 * CH)])

        @pl.when(sid == NS - 1)
        def _():
            pltpu.sync_copy(eb0.at[pl.ds(0, 320 - 2 * CH)],
                            acc.at[pl.ds(zb + 2 * CH, 320 - 2 * CH)])

        plsc.subcore_barrier()

        ebuf = (eb0, eb1)
        sem = (sem0, sem1)

        @pl.loop(0, -(-S_MAX // 2))
        def _(t):
            for b in range(2):
                i = 2 * t + b

                @pl.when(i < n_ch)
                def _():
                    base = (first + i) * CH
                    pltpu.async_copy(e_hbm.at[pl.ds(base, CH)], ebuf[b], sem[b])

            for b in range(2):
                i = 2 * t + b

                @pl.when(i < n_ch)
                def _():
                    base = (first + i) * CH
                    pltpu.make_async_copy(e_hbm.at[pl.ds(base, CH)], ebuf[b], sem[b]).wait()
                    pltpu.sync_copy(ebuf[b], acc.at[didx2.at[i]], add=True)

        plsc.subcore_barrier()

        @pl.when(sid < NS - 1)
        def _():
            pltpu.sync_copy(acc.at[pl.ds(sid * ROWS_S, ROWS_S)],
                            out_hbm.at[pl.ds(lo + sid * ROWS_S, ROWS_S)])

        @pl.when(sid == NS - 1)
        def _():
            pltpu.sync_copy(acc.at[pl.ds(sid * ROWS_S, 320)],
                            out_hbm.at[pl.ds(lo + sid * ROWS_S, 320)])

    return run(e_upd, di2)


def kernel(node_features, edge_features, node_index, edge_index, params):
    nf = node_features.reshape(N_NODES, 128)
    ef = edge_features.reshape(N_EDGES, 16)
    src = edge_index[0].astype(jnp.int32)
    dst = edge_index[1].astype(jnp.int32)
    pad = NCH_PAD * CH - N_EDGES
    si2 = jnp.pad(src, (0, pad)).reshape(NCH_PAD, CH)
    di2 = jnp.pad(dst, (0, pad)).reshape(NCH_PAD, CH)

    def lw(p, i):
        W, b = p["layers"][i]
        return W, b.reshape(1, -1)

    def lng(p):
        g, b = p["ln"]
        return g.reshape(1, -1), b.reshape(1, -1)

    we0, be0 = lw(params["embed"], 0)
    we1, be1 = lw(params["embed"], 1)
    ge, bte = lng(params["embed"])

    wee0, bee0 = lw(params["edge_embed"], 0)
    wee1, bee1 = lw(params["edge_embed"], 1)
    gee, btee = lng(params["edge_embed"])

    wm0, bm0 = lw(params["edge_mlp"], 0)
    wm1, bm1 = lw(params["edge_mlp"], 1)
    gm, btm = lng(params["edge_mlp"])
    w1a, w1b, w1c = wm0[0:H], wm0[H:2 * H], wm0[2 * H:3 * H]

    wn0, bn0 = lw(params["node_mlp"], 0)
    wn1, bn1 = lw(params["node_mlp"], 1)
    gn, btn = lng(params["node_mlp"])
    wn0a, wn0b = wn0[0:H], wn0[H:2 * H]

    wd0, bd0 = lw(params["deembed"], 0)
    wd1, bd1 = lw(params["deembed"], 1)
    gd, btd = lng(params["deembed"])

    ne, p_arr, q_arr = _node_pre(nf, we0, be0, we1, be1, ge, bte, w1a, w1b)
    pg, qg = _sc_gather(p_arr, q_arr, si2, di2)
    e_upd = _edge_fused(ef, pg, qg, wee0, bee0, wee1, bee1, gee, btee,
                        w1c, bm0, wm1, bm1, gm, btm)
    agg = _sc_scatter(e_upd, di2)
    out = _node_post(ne, agg, wn0a, wn0b, bn0, wn1, bn1, gn, btn,
                     wd0, bd0, wd1, bd1, gd, btd)
    return out.reshape(1, N_NODES, H)


# speedup vs baseline: 3.8733x; 1.0141x over previous
import functools

import jax
import jax.numpy as jnp
from jax import lax
from jax.experimental import pallas as pl
from jax.experimental.pallas import tpu as pltpu
from jax.experimental.pallas import tpu_sc as plsc

N_NODES = 10000
N_EDGES = 160000
H = 128

NC = 2
NS = 16
NW = NC * NS
CH = 128
NCH = N_EDGES // CH
NCH_PAD = 1280
NE_PAD = NCH_PAD * CH

G_MAX = NCH_PAD // NW
G_NBUF = 3

S_MAX = NCH_PAD // NC // NS
ZR = 624


def _ln(x, g, b):
    mu = jnp.mean(x, axis=-1, keepdims=True)
    var = jnp.mean((x - mu) ** 2, axis=-1, keepdims=True)
    return (x - mu) * jax.lax.rsqrt(var + 1e-5) * g + b


def _silu(x):
    return x * jax.nn.sigmoid(x)


def _bdot(a, b):
    return jnp.dot(a.astype(jnp.bfloat16), b.astype(jnp.bfloat16),
                   preferred_element_type=jnp.float32)


def _node_pre_body(nf, w0, b0, w1, b1, g, beta, w1a, w1b,
                   ne_out, p_out, q_out):
    x = _silu(jnp.dot(nf[...], w0[...], preferred_element_type=jnp.float32) + b0[...])
    x = jnp.dot(x, w1[...], preferred_element_type=jnp.float32) + b1[...]
    ne = _ln(x, g[...], beta[...])
    ne_out[...] = ne
    p_out[...] = jnp.dot(ne, w1a[...], preferred_element_type=jnp.float32)
    q_out[...] = jnp.dot(ne, w1b[...], preferred_element_type=jnp.float32)


def _node_pre(nf, w0, b0, w1, b1, g, beta, w1a, w1b):
    out = jax.ShapeDtypeStruct((N_NODES, H), jnp.float32)
    return pl.pallas_call(
        _node_pre_body,
        out_shape=[out, out, out],
    )(nf, w0, b0, w1, b1, g, beta, w1a, w1b)


def _edge_fused_body(ef, pg, qg, w0, b0, w1, b1, g, beta, w1c, b1c,
                     wm1, bm1, gm, btm, e_out):
    x = _silu(jnp.dot(ef[...], w0[...], preferred_element_type=jnp.float32) + b0[...])
    x = _bdot(x, w1[...]) + b1[...]
    e = _ln(x, g[...], beta[...])
    c = _bdot(e, w1c[...]) + b1c[...]
    h1 = pg[...] + qg[...] + c
    y = _bdot(_silu(h1), wm1[...]) + bm1[...]
    e_out[...] = _ln(y, gm[...], btm[...])


def _edge_fused(ef, pg, qg, w0, b0, w1, b1, g, beta, w1c, b1c, wm1, bm1, gm, btm):
    R = 4096
    grid = (NE_PAD // R,)
    def rows(i):
        return (i, 0)
    def rep(i):
        return (0, 0)
    return pl.pallas_call(
        _edge_fused_body,
        grid=grid,
        in_specs=[
            pl.BlockSpec((R, 16), rows),
            pl.BlockSpec((R, H), rows),
            pl.BlockSpec((R, H), rows),
            pl.BlockSpec((16, H), rep), pl.BlockSpec((1, H), rep),
            pl.BlockSpec((H, H), rep), pl.BlockSpec((1, H), rep),
            pl.BlockSpec((1, H), rep), pl.BlockSpec((1, H), rep),
            pl.BlockSpec((H, H), rep), pl.BlockSpec((1, H), rep),
            pl.BlockSpec((H, H), rep), pl.BlockSpec((1, H), rep),
            pl.BlockSpec((1, H), rep), pl.BlockSpec((1, H), rep),
        ],
        out_specs=pl.BlockSpec((R, H), rows),
        out_shape=jax.ShapeDtypeStruct((NE_PAD, H), jnp.float32),
    )(ef, pg, qg, w0, b0, w1, b1, g, beta, w1c, b1c, wm1, bm1, gm, btm)


def _node_post_body(ne, pa, wn0a, wn0b, bn0, wn1, bn1, gn, btn,
                    wd0, bd0, wd1, bd1, gd, btd, out):
    agg = pa[0] + pa[1]
    x = _silu(jnp.dot(ne[...], wn0a[...], preferred_element_type=jnp.float32)
              + jnp.dot(agg, wn0b[...], preferred_element_type=jnp.float32)
              + bn0[...])
    x = jnp.dot(x, wn1[...], preferred_element_type=jnp.float32) + bn1[...]
    ne2 = ne[...] + _ln(x, gn[...], btn[...])
    y = _silu(jnp.dot(ne2, wd0[...], preferred_element_type=jnp.float32) + bd0[...])
    y = jnp.dot(y, wd1[...], preferred_element_type=jnp.float32) + bd1[...]
    out[...] = _ln(y, gd[...], btd[...])


def _node_post(ne, pa, wn0a, wn0b, bn0, wn1, bn1, gn, btn,
               wd0, bd0, wd1, bd1, gd, btd):
    return pl.pallas_call(
        _node_post_body,
        out_shape=jax.ShapeDtypeStruct((N_NODES, H), jnp.float32),
    )(ne, pa, wn0a, wn0b, bn0, wn1, bn1, gn, btn,
      wd0, bd0, wd1, bd1, gd, btd)


def _sc_gather(p, q, si2, di2):
    mesh = plsc.VectorSubcoreMesh(core_axis_name="c", subcore_axis_name="s")
    obf = jax.ShapeDtypeStruct((NE_PAD, H), jnp.float32)

    @functools.partial(
        pl.kernel,
        mesh=mesh,
        out_type=[obf, obf],
        scratch_types=(
            [pltpu.VMEM((G_MAX, CH), jnp.int32),
             pltpu.VMEM((G_MAX, CH), jnp.int32)]
            + [pltpu.VMEM((CH, H), jnp.float32) for _ in range(2 * G_NBUF)]
            + [pltpu.SemaphoreType.DMA for _ in range(2 * G_NBUF)]
        ),
    )
    def run(p_hbm, q_hbm, si_hbm, di_hbm, pg_hbm, qg_hbm, sidx, didx, *bufs):
        pbuf = bufs[0:G_NBUF]
        qbuf = bufs[G_NBUF:2 * G_NBUF]
        semg = bufs[2 * G_NBUF:3 * G_NBUF]
        sems = bufs[3 * G_NBUF:4 * G_NBUF]

        wid = lax.axis_index("s") * NC + lax.axis_index("c")
        first = wid * G_MAX
        n_ch = jnp.minimum(jnp.maximum(NCH - first, 0), G_MAX)

        pltpu.sync_copy(si_hbm.at[pl.ds(first, G_MAX)], sidx)
        pltpu.sync_copy(di_hbm.at[pl.ds(first, G_MAX)], didx)

        ngroups = -(-G_MAX // G_NBUF)

        @pl.loop(0, ngroups)
        def _(t):
            for b in range(G_NBUF):
                i = t * G_NBUF + b

                @pl.when(i < n_ch)
                def _():
                    pltpu.async_copy(p_hbm.at[sidx.at[i]], pbuf[b], semg[b])
                    pltpu.async_copy(q_hbm.at[didx.at[i]], qbuf[b], semg[b])

            for b in range(G_NBUF):
                i = t * G_NBUF + b

                @pl.when(i < n_ch)
                def _():
                    base = (first + i) * CH
                    pltpu.make_async_copy(p_hbm.at[sidx.at[i]], pbuf[b], semg[b]).wait()
                    pltpu.make_async_copy(q_hbm.at[didx.at[i]], qbuf[b], semg[b]).wait()
                    pltpu.async_copy(pbuf[b], pg_hbm.at[pl.ds(base, CH)], sems[b])
                    pltpu.async_copy(qbuf[b], qg_hbm.at[pl.ds(base, CH)], sems[b])

            for b in range(G_NBUF):
                i = t * G_NBUF + b

                @pl.when(i < n_ch)
                def _():
                    base = (first + i) * CH
                    pltpu.make_async_copy(pbuf[b], pg_hbm.at[pl.ds(base, CH)], sems[b]).wait()
                    pltpu.make_async_copy(qbuf[b], qg_hbm.at[pl.ds(base, CH)], sems[b]).wait()

    return run(p, q, si2, di2)


def _sc_scatter(e_upd, di2):
    mesh = plsc.VectorSubcoreMesh(core_axis_name="c", subcore_axis_name="s")

    @functools.partial(
        pl.kernel,
        mesh=mesh,
        out_type=jax.ShapeDtypeStruct((NC, N_NODES, H), jnp.float32),
        scratch_types=[
            pltpu.VMEM((S_MAX, CH), jnp.int32),
            pltpu.VMEM((CH, H), jnp.float32),
            pltpu.VMEM((CH, H), jnp.float32),
            pltpu.VMEM_SHARED((N_NODES, H), jnp.float32),
            pltpu.SemaphoreType.DMA,
            pltpu.SemaphoreType.DMA,
        ],
    )
    def run(e_hbm, di_hbm, out_hbm, didx, eb0, eb1, acc, sem0, sem1):
        cid = lax.axis_index("c")
        sid = lax.axis_index("s")
        first = cid * (NCH_PAD // NC) + sid * S_MAX
        n_ch = jnp.minimum(jnp.maximum(NCH - first, 0), S_MAX)

        pltpu.sync_copy(di_hbm.at[pl.ds(first, S_MAX)], didx)

        @pl.loop(0, CH)
        def _(r):
            for j in range(H // 16):
                eb0[r, pl.ds(j * 16, 16)] = jnp.zeros((16,), jnp.float32)

        zb = sid * ZR

        @pl.when(sid < NS - 1)
        def _():
            for k in range(4):
                pltpu.sync_copy(eb0, acc.at[pl.ds(zb + k * CH, CH)])
            pltpu.sync_copy(eb0.at[pl.ds(0, ZR - 4 * CH)],
                            acc.at[pl.ds(zb + 4 * CH, ZR - 4 * CH)])

        @pl.when(sid == NS - 1)
        def _():
            for k in range(5):
                pltpu.sync_copy(eb0, acc.at[pl.ds(zb + k * CH, CH)])

        plsc.subcore_barrier()

        ebuf = (eb0, eb1)
        sem = (sem0, sem1)

        @pl.loop(0, S_MAX // 2)
        def _(t):
            for b in range(2):
                i = 2 * t + b

                @pl.when(i < n_ch)
                def _():
                    base = (first + i) * CH
                    pltpu.async_copy(e_hbm.at[pl.ds(base, CH)], ebuf[b], sem[b])

            for b in range(2):
                i = 2 * t + b

                @pl.when(i < n_ch)
                def _():
                    base = (first + i) * CH
                    pltpu.make_async_copy(e_hbm.at[pl.ds(base, CH)], ebuf[b], sem[b]).wait()
                    pltpu.sync_copy(ebuf[b], acc.at[didx.at[i]], add=True)

        plsc.subcore_barrier()

        @pl.when(sid < NS - 1)
        def _():
            pltpu.sync_copy(acc.at[pl.ds(sid * ZR, ZR)],
                            out_hbm.at[cid, pl.ds(sid * ZR, ZR)])

        @pl.when(sid == NS - 1)
        def _():
            pltpu.sync_copy(acc.at[pl.ds(sid * ZR, 640)],
                            out_hbm.at[cid, pl.ds(sid * ZR, 640)])

    return run(e_upd, di2)


def kernel(node_features, edge_features, node_index, edge_index, params):
    nf = node_features.reshape(N_NODES, 128)
    ef = edge_features.reshape(N_EDGES, 16)
    src = edge_index[0].astype(jnp.int32)
    dst = edge_index[1].astype(jnp.int32)
    pad = NE_PAD - N_EDGES
    si2 = jnp.pad(src, (0, pad)).reshape(NCH_PAD, CH)
    di2 = jnp.pad(dst, (0, pad)).reshape(NCH_PAD, CH)
    efp = jnp.pad(ef, ((0, pad), (0, 0)))

    def lw(p, i):
        W, b = p["layers"][i]
        return W, b.reshape(1, -1)

    def lng(p):
        g, b = p["ln"]
        return g.reshape(1, -1), b.reshape(1, -1)

    we0, be0 = lw(params["embed"], 0)
    we1, be1 = lw(params["embed"], 1)
    ge, bte = lng(params["embed"])

    wee0, bee0 = lw(params["edge_embed"], 0)
    wee1, bee1 = lw(params["edge_embed"], 1)
    gee, btee = lng(params["edge_embed"])

    wm0, bm0 = lw(params["edge_mlp"], 0)
    wm1, bm1 = lw(params["edge_mlp"], 1)
    gm, btm = lng(params["edge_mlp"])
    w1a, w1b, w1c = wm0[0:H], wm0[H:2 * H], wm0[2 * H:3 * H]

    wn0, bn0 = lw(params["node_mlp"], 0)
    wn1, bn1 = lw(params["node_mlp"], 1)
    gn, btn = lng(params["node_mlp"])
    wn0a, wn0b = wn0[0:H], wn0[H:2 * H]

    wd0, bd0 = lw(params["deembed"], 0)
    wd1, bd1 = lw(params["deembed"], 1)
    gd, btd = lng(params["deembed"])

    ne, p_arr, q_arr = _node_pre(nf, we0, be0, we1, be1, ge, bte, w1a, w1b)
    pg, qg = _sc_gather(p_arr, q_arr, si2, di2)
    e_upd = _edge_fused(efp, pg, qg, wee0, bee0, wee1, bee1, gee, btee,
                        w1c, bm0, wm1, bm1, gm, btm)
    partials = _sc_scatter(e_upd, di2)
    out = _node_post(ne, partials, wn0a, wn0b, bn0, wn1, bn1, gn, btn,
                     wd0, bd0, wd1, bd1, gd, btd)
    return out.reshape(1, N_NODES, H)
